# baseline (speedup 1.0000x reference)
import jax
import jax.numpy as jnp
from jax.experimental import pallas as pl
from jax.experimental.pallas import tpu as pltpu


CHUNK = 2048
NBUF = 3


def _mlp_kernel(x_hbm, w_ref, b_ref, o_hbm, xbuf, obuf, load_sem, store_sem):
    n = x_hbm.shape[0]
    nchunks = n // CHUNK

    def load(i, slot):
        return pltpu.make_async_copy(
            x_hbm.at[pl.ds(i * CHUNK, CHUNK), :], xbuf.at[slot],
            load_sem.at[slot])

    def store(i, slot):
        return pltpu.make_async_copy(
            obuf.at[slot], o_hbm.at[pl.ds(i * CHUNK, CHUNK), :],
            store_sem.at[slot])

    for k in range(min(NBUF, nchunks)):
        load(k, k).start()

    for i in range(nchunks):
        slot = i % NBUF
        load(i, slot).wait()
        if i >= NBUF:
            store(i - NBUF, slot).wait()
        acc = jax.lax.dot_general(
            xbuf[slot], w_ref[...],
            dimension_numbers=(((1,), (1,)), ((), ())),
            preferred_element_type=jnp.float32,
        )
        obuf[slot] = acc + b_ref[...]
        store(i, slot).start()
        if i + NBUF < nchunks:
            load(i + NBUF, slot).start()

    for i in range(max(nchunks - NBUF, 0), nchunks):
        store(i, i % NBUF).wait()


@jax.jit
def kernel(x, W, b):
    n, hidden = x.shape
    out_dim = W.shape[0]
    b2 = b.reshape(1, out_dim)
    return pl.pallas_call(
        _mlp_kernel,
        in_specs=[
            pl.BlockSpec(memory_space=pl.ANY),
            pl.BlockSpec(memory_space=pltpu.VMEM),
            pl.BlockSpec(memory_space=pltpu.VMEM),
        ],
        out_specs=pl.BlockSpec(memory_space=pl.ANY),
        out_shape=jax.ShapeDtypeStruct((n, out_dim), jnp.float32),
        scratch_shapes=[
            pltpu.VMEM((NBUF, CHUNK, out_dim), jnp.float32),
            pltpu.VMEM((NBUF, CHUNK, out_dim), jnp.float32),
            pltpu.SemaphoreType.DMA((NBUF,)),
            pltpu.SemaphoreType.DMA((NBUF,)),
        ],
    )(x, W, b2)


# speedup vs baseline: 1.0016x; 1.0016x over previous
import jax
import jax.numpy as jnp
from jax.experimental import pallas as pl
from jax.experimental.pallas import tpu as pltpu


CHUNK = 1024
NBUF = 4


def _mlp_kernel(x_hbm, w_ref, b_ref, o_hbm, xbuf, obuf, load_sem, store_sem):
    n = x_hbm.shape[0]
    nchunks = n // CHUNK

    half = CHUNK // 2

    def load_halves(i, slot):
        return [
            pltpu.make_async_copy(
                x_hbm.at[pl.ds(i * CHUNK + h * half, half), :],
                xbuf.at[slot, pl.ds(h * half, half), :],
                load_sem.at[slot])
            for h in range(2)
        ]

    def store_halves(i, slot):
        return [
            pltpu.make_async_copy(
                obuf.at[slot, pl.ds(h * half, half), :],
                o_hbm.at[pl.ds(i * CHUNK + h * half, half), :],
                store_sem.at[slot])
            for h in range(2)
        ]

    def load(i, slot):
        class _Pair:
            def start(self):
                for c in load_halves(i, slot):
                    c.start()
            def wait(self):
                for c in load_halves(i, slot):
                    c.wait()
        return _Pair()

    def store(i, slot):
        class _Pair:
            def start(self):
                for c in store_halves(i, slot):
                    c.start()
            def wait(self):
                for c in store_halves(i, slot):
                    c.wait()
        return _Pair()

    for k in range(min(NBUF, nchunks)):
        load(k, k).start()

    for i in range(nchunks):
        slot = i % NBUF
        load(i, slot).wait()
        if i >= NBUF:
            store(i - NBUF, slot).wait()
        acc = jax.lax.dot_general(
            xbuf[slot], w_ref[...],
            dimension_numbers=(((1,), (1,)), ((), ())),
            preferred_element_type=jnp.float32,
        )
        obuf[slot] = acc + b_ref[...]
        store(i, slot).start()
        if i + NBUF < nchunks:
            load(i + NBUF, slot).start()

    for i in range(max(nchunks - NBUF, 0), nchunks):
        store(i, i % NBUF).wait()


@jax.jit
def kernel(x, W, b):
    n, hidden = x.shape
    out_dim = W.shape[0]
    b2 = b.reshape(1, out_dim)
    return pl.pallas_call(
        _mlp_kernel,
        in_specs=[
            pl.BlockSpec(memory_space=pl.ANY),
            pl.BlockSpec(memory_space=pltpu.VMEM),
            pl.BlockSpec(memory_space=pltpu.VMEM),
        ],
        out_specs=pl.BlockSpec(memory_space=pl.ANY),
        out_shape=jax.ShapeDtypeStruct((n, out_dim), jnp.float32),
        scratch_shapes=[
            pltpu.VMEM((NBUF, CHUNK, out_dim), jnp.float32),
            pltpu.VMEM((NBUF, CHUNK, out_dim), jnp.float32),
            pltpu.SemaphoreType.DMA((NBUF,)),
            pltpu.SemaphoreType.DMA((NBUF,)),
        ],
    )(x, W, b2)


# speedup vs baseline: 1.1011x; 1.0993x over previous
import jax
import jax.numpy as jnp
from jax.experimental import pallas as pl
from jax.experimental.pallas import tpu as pltpu


CHUNK = 1024
NBUF = 4


def _mlp_kernel(x_hbm, w_ref, b_ref, o_hbm, xbuf, obuf, load_sem, store_sem):
    n = x_hbm.shape[0]
    nchunks = n // CHUNK

    def load(i, slot):
        return pltpu.make_async_copy(
            x_hbm.at[pl.ds(i * CHUNK, CHUNK), :], xbuf.at[slot],
            load_sem.at[slot])

    def store(i, slot):
        return pltpu.make_async_copy(
            obuf.at[slot], o_hbm.at[pl.ds(i * CHUNK, CHUNK), :],
            store_sem.at[slot])

    for k in range(min(NBUF, nchunks)):
        load(k, k).start()

    for i in range(nchunks):
        slot = i % NBUF
        load(i, slot).wait()
        if i >= NBUF:
            store(i - NBUF, slot).wait()
        acc = jax.lax.dot_general(
            xbuf[slot], w_ref[...],
            dimension_numbers=(((1,), (1,)), ((), ())),
            preferred_element_type=jnp.float32,
            precision=jax.lax.Precision.DEFAULT,
        )
        obuf[slot] = acc + b_ref[...]
        store(i, slot).start()
        if i + NBUF < nchunks:
            load(i + NBUF, slot).start()

    for i in range(max(nchunks - NBUF, 0), nchunks):
        store(i, i % NBUF).wait()


@jax.jit
def kernel(x, W, b):
    n, hidden = x.shape
    out_dim = W.shape[0]
    b2 = b.reshape(1, out_dim)
    return pl.pallas_call(
        _mlp_kernel,
        in_specs=[
            pl.BlockSpec(memory_space=pl.ANY),
            pl.BlockSpec(memory_space=pltpu.VMEM),
            pl.BlockSpec(memory_space=pltpu.VMEM),
        ],
        out_specs=pl.BlockSpec(memory_space=pl.ANY),
        out_shape=jax.ShapeDtypeStruct((n, out_dim), jnp.float32),
        scratch_shapes=[
            pltpu.VMEM((NBUF, CHUNK, out_dim), jnp.float32),
            pltpu.VMEM((NBUF, CHUNK, out_dim), jnp.float32),
            pltpu.SemaphoreType.DMA((NBUF,)),
            pltpu.SemaphoreType.DMA((NBUF,)),
        ],
    )(x, W, b2)


# speedup vs baseline: 1.1019x; 1.0008x over previous
import jax
import jax.numpy as jnp
from jax.experimental import pallas as pl
from jax.experimental.pallas import tpu as pltpu


CHUNK = 1024
NBUF = 4


def _mlp_kernel(x_hbm, w_ref, b_ref, o_hbm, xbuf, obuf, wbuf, load_sem, store_sem):
    n = x_hbm.shape[0]
    nchunks = n // CHUNK
    wbuf[...] = w_ref[...].astype(jnp.bfloat16)

    def load(i, slot):
        return pltpu.make_async_copy(
            x_hbm.at[pl.ds(i * CHUNK, CHUNK), :], xbuf.at[slot],
            load_sem.at[slot])

    def store(i, slot):
        return pltpu.make_async_copy(
            obuf.at[slot], o_hbm.at[pl.ds(i * CHUNK, CHUNK), :],
            store_sem.at[slot])

    for k in range(min(NBUF, nchunks)):
        load(k, k).start()

    for i in range(nchunks):
        slot = i % NBUF
        load(i, slot).wait()
        if i >= NBUF:
            store(i - NBUF, slot).wait()
        acc = jax.lax.dot_general(
            xbuf[slot].astype(jnp.bfloat16), wbuf[...],
            dimension_numbers=(((1,), (1,)), ((), ())),
            preferred_element_type=jnp.float32,
        )
        obuf[slot] = acc + b_ref[...]
        store(i, slot).start()
        if i + NBUF < nchunks:
            load(i + NBUF, slot).start()

    for i in range(max(nchunks - NBUF, 0), nchunks):
        store(i, i % NBUF).wait()


@jax.jit
def kernel(x, W, b):
    n, hidden = x.shape
    out_dim = W.shape[0]
    b2 = b.reshape(1, out_dim)
    return pl.pallas_call(
        _mlp_kernel,
        in_specs=[
            pl.BlockSpec(memory_space=pl.ANY),
            pl.BlockSpec(memory_space=pltpu.VMEM),
            pl.BlockSpec(memory_space=pltpu.VMEM),
        ],
        out_specs=pl.BlockSpec(memory_space=pl.ANY),
        out_shape=jax.ShapeDtypeStruct((n, out_dim), jnp.float32),
        scratch_shapes=[
            pltpu.VMEM((NBUF, CHUNK, out_dim), jnp.float32),
            pltpu.VMEM((NBUF, CHUNK, out_dim), jnp.float32),
            pltpu.VMEM((out_dim, hidden), jnp.bfloat16),
            pltpu.SemaphoreType.DMA((NBUF,)),
            pltpu.SemaphoreType.DMA((NBUF,)),
        ],
    )(x, W, b2)


# speedup vs baseline: 1.1116x; 1.0088x over previous
import jax
import jax.numpy as jnp
from jax.experimental import pallas as pl
from jax.experimental.pallas import tpu as pltpu


CHUNK = 1024
NBUF = 4
HALF = CHUNK // 2


def _mlp_kernel(x_hbm, w_ref, b_ref, o_hbm, xbuf, obuf, load_sem, store_sem):
    n = x_hbm.shape[0]
    nchunks = n // CHUNK

    def load(i, slot):
        return pltpu.make_async_copy(
            x_hbm.at[pl.ds(i * CHUNK, CHUNK), :], xbuf.at[slot],
            load_sem.at[slot])

    def store_half(i, slot, h):
        return pltpu.make_async_copy(
            obuf.at[slot, pl.ds(h * HALF, HALF), :],
            o_hbm.at[pl.ds(i * CHUNK + h * HALF, HALF), :],
            store_sem.at[slot])

    def wait_store(i, slot):
        store_half(i, slot, 0).wait()
        store_half(i, slot, 1).wait()

    for k in range(min(NBUF, nchunks)):
        load(k, k).start()

    for i in range(nchunks):
        slot = i % NBUF
        load(i, slot).wait()
        if i >= NBUF:
            wait_store(i - NBUF, slot)
        for h in range(2):
            acc = jax.lax.dot_general(
                xbuf[slot, pl.ds(h * HALF, HALF), :], w_ref[...],
                dimension_numbers=(((1,), (1,)), ((), ())),
                preferred_element_type=jnp.float32,
            )
            obuf[slot, pl.ds(h * HALF, HALF), :] = acc + b_ref[...]
            store_half(i, slot, h).start()
        if i + NBUF < nchunks:
            load(i + NBUF, slot).start()

    for i in range(max(nchunks - NBUF, 0), nchunks):
        wait_store(i, i % NBUF)


@jax.jit
def kernel(x, W, b):
    n, hidden = x.shape
    out_dim = W.shape[0]
    b2 = b.reshape(1, out_dim)
    return pl.pallas_call(
        _mlp_kernel,
        in_specs=[
            pl.BlockSpec(memory_space=pl.ANY),
            pl.BlockSpec(memory_space=pltpu.VMEM),
            pl.BlockSpec(memory_space=pltpu.VMEM),
        ],
        out_specs=pl.BlockSpec(memory_space=pl.ANY),
        out_shape=jax.ShapeDtypeStruct((n, out_dim), jnp.float32),
        scratch_shapes=[
            pltpu.VMEM((NBUF, CHUNK, hidden), jnp.float32),
            pltpu.VMEM((NBUF, CHUNK, out_dim), jnp.float32),
            pltpu.SemaphoreType.DMA((NBUF,)),
            pltpu.SemaphoreType.DMA((NBUF,)),
        ],
    )(x, W, b2)
